# baseline (speedup 1.0000x reference)
import functools

import jax
import jax.numpy as jnp
from jax import lax
from jax.experimental import pallas as pl
from jax.experimental.pallas import tpu as pltpu
from jax.experimental.pallas import tpu_sc as plsc

VOCAB = 1000000
EMBED = 64
BATCH = 4096
SEQ = 200
NROWS = BATCH * SEQ
NC = 2
NS = 16
NW = NC * NS
ROWS_PER_W = NROWS // NW
IDXW = 128
CH = 512
NJ = CH // IDXW
NCHUNKS = ROWS_PER_W // CH
IROWS_PER_W = ROWS_PER_W // IDXW

_mesh = plsc.VectorSubcoreMesh(core_axis_name="c", subcore_axis_name="s")


@functools.partial(
    pl.kernel,
    mesh=_mesh,
    out_type=jax.ShapeDtypeStruct((NROWS, EMBED), jnp.float32),
    scratch_types=[
        pltpu.VMEM((NJ, IDXW), jnp.int32),
        pltpu.VMEM((CH, EMBED), jnp.float32),
        pltpu.SemaphoreType.DMA,
    ],
    compiler_params=pltpu.CompilerParams(
        needs_layout_passes=False, use_tc_tiling_on_sc=False
    ),
)
def _embed(idx_hbm, table_hbm, out_hbm, idx_v, rows_v, gsem):
    wid = lax.axis_index("s") * NC + lax.axis_index("c")
    row0 = wid * ROWS_PER_W
    irow0 = wid * IROWS_PER_W

    zeros16 = jnp.zeros((16,), jnp.float32)
    lane = lax.iota(jnp.int32, 16)

    def chunk_body(g, carry):
        rbase = row0 + g * CH
        ibase = irow0 + g * NJ
        pltpu.sync_copy(idx_hbm.at[pl.ds(ibase, NJ), :], idx_v)
        for j in range(NJ):
            pltpu.async_copy(
                table_hbm.at[idx_v.at[j]],
                rows_v.at[pl.ds(j * IDXW, IDXW), :],
                gsem,
            )
        for j in range(NJ):
            pltpu.make_async_copy(
                table_hbm.at[idx_v.at[j]],
                rows_v.at[pl.ds(j * IDXW, IDXW), :],
                gsem,
            ).wait()

        def fix_body(i, fcarry):
            j = jnp.right_shift(i, 3)
            t = jnp.bitwise_and(i, 7)
            idxv = idx_v[j, pl.ds(t * 16, 16)]
            m = idxv == 0

            nzero = plsc.all_reduce_population_count(m)

            @pl.when(nzero[0] > 0)
            def _zero_rows():
                rows16 = i * 16 + lane
                for c in range(EMBED):
                    plsc.store_scatter(
                        rows_v,
                        [rows16, jnp.full((16,), c, jnp.int32)],
                        zeros16,
                        mask=m,
                    )

            return fcarry

        lax.fori_loop(0, CH // 16, fix_body, 0)

        pltpu.sync_copy(rows_v, out_hbm.at[pl.ds(rbase, CH), :])
        return carry

    lax.fori_loop(0, NCHUNKS, chunk_body, 0)


def kernel(x, E):
    xi = x.astype(jnp.int32).reshape(NROWS // IDXW, IDXW)
    out = _embed(xi, E)
    return out.reshape(BATCH, SEQ, EMBED)


# speedup vs baseline: 1.0726x; 1.0726x over previous
import functools

import jax
import jax.numpy as jnp
from jax import lax
from jax.experimental import pallas as pl
from jax.experimental.pallas import tpu as pltpu
from jax.experimental.pallas import tpu_sc as plsc

VOCAB = 1000000
EMBED = 64
BATCH = 4096
SEQ = 200
NROWS = BATCH * SEQ
NC = 2
NS = 16
NW = NC * NS
ROWS_PER_W = NROWS // NW
IDXW = 128
CH = 512
NJ = CH // IDXW
NCHUNKS = ROWS_PER_W // CH
IROWS_PER_W = ROWS_PER_W // IDXW
NB = 3

_mesh = plsc.VectorSubcoreMesh(core_axis_name="c", subcore_axis_name="s")


@functools.partial(
    pl.kernel,
    mesh=_mesh,
    out_type=jax.ShapeDtypeStruct((NROWS, EMBED), jnp.float32),
    scratch_types=[
        pltpu.VMEM((NB, NJ, IDXW), jnp.int32),
        pltpu.VMEM((NB, CH, EMBED), jnp.float32),
        pltpu.SemaphoreType.DMA((NB,)),
        pltpu.SemaphoreType.DMA((NB,)),
        pltpu.SemaphoreType.DMA((NB,)),
    ],
    compiler_params=pltpu.CompilerParams(
        needs_layout_passes=False, use_tc_tiling_on_sc=False
    ),
)
def _embed(idx_hbm, table_hbm, out_hbm, idx_v, rows_v, isem, gsem, wsem):
    wid = lax.axis_index("s") * NC + lax.axis_index("c")
    row0 = wid * ROWS_PER_W
    irow0 = wid * IROWS_PER_W

    zeros16 = jnp.zeros((16,), jnp.float32)
    lane = lax.iota(jnp.int32, 16)

    def fire_idx(g, b):
        pltpu.async_copy(
            idx_hbm.at[pl.ds(irow0 + g * NJ, NJ), :], idx_v.at[b], isem.at[b]
        )

    def wait_idx(b):
        pltpu.make_async_copy(
            idx_hbm.at[pl.ds(irow0, NJ), :], idx_v.at[b], isem.at[b]
        ).wait()

    def fire_gathers(b):
        for j in range(NJ):
            pltpu.async_copy(
                table_hbm.at[idx_v.at[b, j]],
                rows_v.at[b, pl.ds(j * IDXW, IDXW), :],
                gsem.at[b],
            )

    def wait_gathers(b):
        for j in range(NJ):
            pltpu.make_async_copy(
                table_hbm.at[idx_v.at[b, j]],
                rows_v.at[b, pl.ds(j * IDXW, IDXW), :],
                gsem.at[b],
            ).wait()

    def fire_write(g, b):
        pltpu.async_copy(
            rows_v.at[b], out_hbm.at[pl.ds(row0 + g * CH, CH), :], wsem.at[b]
        )

    def wait_write(b):
        pltpu.make_async_copy(
            rows_v.at[b], out_hbm.at[pl.ds(row0, CH), :], wsem.at[b]
        ).wait()

    def fix(b):
        def fix_body(i, fcarry):
            j = jnp.right_shift(i, 3)
            t = jnp.bitwise_and(i, 7)
            idxv = idx_v[b, j, pl.ds(t * 16, 16)]
            m = idxv == 0
            nzero = plsc.all_reduce_population_count(m)

            @pl.when(nzero[0] > 0)
            def _zero_rows():
                rows16 = i * 16 + lane
                for c in range(EMBED):
                    plsc.store_scatter(
                        rows_v.at[b],
                        [rows16, jnp.full((16,), c, jnp.int32)],
                        zeros16,
                        mask=m,
                    )

            return fcarry

        lax.fori_loop(0, CH // 16, fix_body, 0)

    for b in range(NB):
        fire_idx(b, b)
    wait_idx(0)
    fire_gathers(0)
    wait_idx(1)
    fire_gathers(1)

    def block_body(p, carry):
        for b in range(NB):
            g = p * NB + b
            g2 = g + 2
            b2 = (b + 2) % NB

            @pl.when(g2 < NCHUNKS)
            def _ahead():
                @pl.when(g2 >= NB)
                def _reuse_wait():
                    wait_write(b2)

                wait_idx(b2)
                fire_gathers(b2)

            @pl.when(g < NCHUNKS)
            def _iter():
                wait_gathers(b)
                fix(b)
                fire_write(g, b)

                @pl.when(g + NB < NCHUNKS)
                def _prefetch_idx():
                    fire_idx(g + NB, b)

        return carry

    lax.fori_loop(0, (NCHUNKS + NB - 1) // NB, block_body, 0)

    for b in range(NB):
        wait_write(b)


def kernel(x, E):
    xi = x.astype(jnp.int32).reshape(NROWS // IDXW, IDXW)
    out = _embed(xi, E)
    return out.reshape(BATCH, SEQ, EMBED)
